# Initial kernel scaffold; baseline (speedup 1.0000x reference)
import jax
import jax.numpy as jnp
from jax.experimental import pallas as pl


def kernel(x, edge_index, weight, bias):
    raise NotImplementedError("write your pallas kernel here")


# speedup vs baseline: 3.9578x; 3.9578x over previous
import functools

import jax
import jax.numpy as jnp
from jax import lax
from jax.experimental import pallas as pl
from jax.experimental.pallas import tpu as pltpu
from jax.experimental.pallas import tpu_sc as plsc

NC = 2
NS = 16
NW = NC * NS
K = 128


def _cdiv(a, b):
    return (a + b - 1) // b


def _stage_blocks(rpt):
    return [(r * K, min(K, rpt - r * K)) for r in range(_cdiv(rpt, K))]


@functools.lru_cache(maxsize=None)
def _sc_aggregate(N, D, EP, NP):
    CH = EP // (NW * K)
    RPT = NP // NS

    mesh = plsc.VectorSubcoreMesh(core_axis_name="c", subcore_axis_name="s")

    @functools.partial(
        pl.kernel,
        mesh=mesh,
        out_type=jax.ShapeDtypeStruct((NC * NP, D), jnp.float32),
        scratch_types=[
            pltpu.VMEM((K,), jnp.int32),
            pltpu.VMEM((K,), jnp.int32),
            pltpu.VMEM((K, D), jnp.float32),
            pltpu.VMEM_SHARED((NP, D), jnp.float32),
            pltpu.SemaphoreType.DMA,
        ],
    )
    def agg(x_hbm, srcp_hbm, dstp_hbm, zn_hbm, nei_out,
            src_v, dst_v, rows_v, acc, sem):
        cid = lax.axis_index("c")
        sid = lax.axis_index("s")
        wid = cid * NS + sid

        rows0 = sid * RPT
        pltpu.sync_copy(zn_hbm, rows_v)
        for off, sz in _stage_blocks(RPT):
            pltpu.sync_copy(rows_v.at[pl.ds(0, sz)],
                            acc.at[pl.ds(rows0 + off, sz)])
        plsc.subcore_barrier()

        def chunk(g, carry):
            base = (wid * CH + g) * K
            pltpu.sync_copy(srcp_hbm.at[pl.ds(base, K)], src_v)
            pltpu.sync_copy(dstp_hbm.at[pl.ds(base, K)], dst_v)
            pltpu.async_copy(x_hbm.at[src_v], rows_v, sem).wait()
            pltpu.sync_copy(rows_v, acc.at[dst_v], add=True)
            return carry

        lax.fori_loop(0, CH, chunk, 0)
        plsc.subcore_barrier()

        out0 = cid * NP + rows0
        for off, sz in _stage_blocks(RPT):
            pltpu.sync_copy(acc.at[pl.ds(rows0 + off, sz)],
                            rows_v.at[pl.ds(0, sz)])
            pltpu.sync_copy(rows_v.at[pl.ds(0, sz)],
                            nei_out.at[pl.ds(out0 + off, sz)])

    return agg


@functools.lru_cache(maxsize=None)
def _sc_degree(D, EP, NP):
    CH = EP // (NW * K)
    RPT = NP // NS

    mesh = plsc.VectorSubcoreMesh(core_axis_name="c", subcore_axis_name="s")

    @functools.partial(
        pl.kernel,
        mesh=mesh,
        out_type=jax.ShapeDtypeStruct((NC * NP, D), jnp.float32),
        scratch_types=[
            pltpu.VMEM((K,), jnp.int32),
            pltpu.VMEM((K, D), jnp.float32),
            pltpu.VMEM_SHARED((NP, D), jnp.float32),
        ],
    )
    def deg(dstp_hbm, zn_hbm, ones_hbm, deg_out,
            dst_v, st_v, dacc):
        cid = lax.axis_index("c")
        sid = lax.axis_index("s")
        wid = cid * NS + sid

        rows0 = sid * RPT
        pltpu.sync_copy(zn_hbm, st_v)
        for off, sz in _stage_blocks(RPT):
            pltpu.sync_copy(st_v.at[pl.ds(0, sz)],
                            dacc.at[pl.ds(rows0 + off, sz)])
        pltpu.sync_copy(ones_hbm, st_v)
        plsc.subcore_barrier()

        def chunk(g, carry):
            base = (wid * CH + g) * K
            pltpu.sync_copy(dstp_hbm.at[pl.ds(base, K)], dst_v)
            pltpu.sync_copy(st_v, dacc.at[dst_v], add=True)
            return carry

        lax.fori_loop(0, CH, chunk, 0)
        plsc.subcore_barrier()

        out0 = cid * NP + rows0
        for off, sz in _stage_blocks(RPT):
            pltpu.sync_copy(dacc.at[pl.ds(rows0 + off, sz)],
                            st_v.at[pl.ds(0, sz)])
            pltpu.sync_copy(st_v.at[pl.ds(0, sz)],
                            deg_out.at[pl.ds(out0 + off, sz)])

    return deg


def _finalize_body(x_ref, nei_ref, deg_ref, w_ref, b_ref, o_ref, *, D):
    nei = nei_ref[0] + nei_ref[1]
    deg = deg_ref[0, :, 0:1] + deg_ref[1, :, 0:1]
    mean = nei / jnp.maximum(deg, 1.0)
    o_ref[...] = (
        jnp.dot(x_ref[...], w_ref[0:D, :], preferred_element_type=jnp.float32)
        + jnp.dot(mean, w_ref[D:, :], preferred_element_type=jnp.float32)
        + b_ref[...]
    )


def kernel(x, edge_index, weight, bias):
    N, D = x.shape
    E = edge_index.shape[1]
    OUT = weight.shape[1]

    CH = _cdiv(E, NW * K)
    EP = NW * K * CH
    NP = _cdiv(N + 1, 128) * 128

    src = edge_index[0]
    dst = edge_index[1]
    pad = EP - E
    srcp = jnp.concatenate([src, jnp.zeros((pad,), jnp.int32)])
    dstp = jnp.concatenate([dst, jnp.full((pad,), N, jnp.int32)])
    zn = jnp.zeros((K, D), jnp.float32)
    ones_k = jnp.ones((K, D), jnp.float32)

    nei_flat = _sc_aggregate(N, D, EP, NP)(x, srcp, dstp, zn)
    deg_flat = _sc_degree(D, EP, NP)(dstp, zn, ones_k)
    nei_p = nei_flat.reshape(NC, NP, D)
    deg_p = deg_flat.reshape(NC, NP, D)

    BR = 2000
    out = pl.pallas_call(
        functools.partial(_finalize_body, D=D),
        grid=(N // BR,),
        in_specs=[
            pl.BlockSpec((BR, D), lambda i: (i, 0)),
            pl.BlockSpec((NC, BR, D), lambda i: (0, i, 0)),
            pl.BlockSpec((NC, BR, D), lambda i: (0, i, 0)),
            pl.BlockSpec((2 * D, OUT), lambda i: (0, 0)),
            pl.BlockSpec((1, OUT), lambda i: (0, 0)),
        ],
        out_specs=pl.BlockSpec((BR, OUT), lambda i: (i, 0)),
        out_shape=jax.ShapeDtypeStruct((N, OUT), jnp.float32),
    )(x, nei_p, deg_p, weight, bias.reshape(1, OUT))
    return out


# speedup vs baseline: 4.4826x; 1.1326x over previous
import functools

import jax
import jax.numpy as jnp
from jax import lax
from jax.experimental import pallas as pl
from jax.experimental.pallas import tpu as pltpu
from jax.experimental.pallas import tpu_sc as plsc

NC = 2
NS = 16
NW = NC * NS
K = 128


def _cdiv(a, b):
    return (a + b - 1) // b


def _stage_blocks(rpt, blk=K):
    return [(r * blk, min(blk, rpt - r * blk)) for r in range(_cdiv(rpt, blk))]


KG = 64


@functools.lru_cache(maxsize=None)
def _sc_aggregate(N, D, EP, NP):
    CH = EP // (NW * KG)
    RPT = NP // NS

    mesh = plsc.VectorSubcoreMesh(core_axis_name="c", subcore_axis_name="s")

    @functools.partial(
        pl.kernel,
        mesh=mesh,
        out_type=jax.ShapeDtypeStruct((NC * NP, D), jnp.float32),
        scratch_types=[
            pltpu.VMEM((KG,), jnp.int32),
            pltpu.VMEM((KG,), jnp.int32),
            pltpu.VMEM((KG,), jnp.int32),
            pltpu.VMEM((KG,), jnp.int32),
            pltpu.VMEM((KG, D), jnp.float32),
            pltpu.VMEM((KG, D), jnp.float32),
            pltpu.VMEM_SHARED((NP, D), jnp.float32),
            pltpu.SemaphoreType.DMA,
            pltpu.SemaphoreType.DMA,
        ],
    )
    def agg(x_hbm, srcp_hbm, dstp_hbm, zn_hbm, nei_out,
            src_v0, src_v1, dst_v0, dst_v1, rows_v0, rows_v1,
            acc, sem0, sem1):
        cid = lax.axis_index("c")
        sid = lax.axis_index("s")
        wid = cid * NS + sid
        srcs = (src_v0, src_v1)
        dsts = (dst_v0, dst_v1)
        rows = (rows_v0, rows_v1)
        sems = (sem0, sem1)

        rows0 = sid * RPT
        pltpu.sync_copy(zn_hbm, rows_v0)
        for off, sz in _stage_blocks(RPT, KG):
            pltpu.sync_copy(rows_v0.at[pl.ds(0, sz)],
                            acc.at[pl.ds(rows0 + off, sz)])
        plsc.subcore_barrier()

        base0 = wid * CH * KG

        pltpu.sync_copy(srcp_hbm.at[pl.ds(base0, KG)], src_v0)
        pltpu.sync_copy(dstp_hbm.at[pl.ds(base0, KG)], dst_v0)
        pltpu.async_copy(x_hbm.at[src_v0], rows_v0, sem0)

        def pair(g2, carry):
            for b in (0, 1):
                g = g2 * 2 + b
                nb = 1 - b

                @pl.when(g + 1 < CH)
                def _prefetch():
                    nxt = base0 + (g + 1) * KG
                    pltpu.sync_copy(srcp_hbm.at[pl.ds(nxt, KG)], srcs[nb])
                    pltpu.sync_copy(dstp_hbm.at[pl.ds(nxt, KG)], dsts[nb])
                    pltpu.async_copy(x_hbm.at[srcs[nb]], rows[nb], sems[nb])

                pltpu.make_async_copy(x_hbm.at[srcs[b]], rows[b], sems[b]).wait()
                pltpu.sync_copy(rows[b], acc.at[dsts[b]], add=True)
            return carry

        lax.fori_loop(0, CH // 2, pair, 0)
        plsc.subcore_barrier()

        out0 = cid * NP + rows0
        for off, sz in _stage_blocks(RPT, KG):
            buf = rows[(off // KG) % 2]
            pltpu.sync_copy(acc.at[pl.ds(rows0 + off, sz)],
                            buf.at[pl.ds(0, sz)])
            pltpu.sync_copy(buf.at[pl.ds(0, sz)],
                            nei_out.at[pl.ds(out0 + off, sz)])

    return agg


@functools.lru_cache(maxsize=None)
def _sc_degree(D, EP, NP):
    CH = EP // (NW * K)
    RPT = NP // NS

    mesh = plsc.VectorSubcoreMesh(core_axis_name="c", subcore_axis_name="s")

    @functools.partial(
        pl.kernel,
        mesh=mesh,
        out_type=jax.ShapeDtypeStruct((NC * NP, D), jnp.float32),
        scratch_types=[
            pltpu.VMEM((K,), jnp.int32),
            pltpu.VMEM((K, D), jnp.float32),
            pltpu.VMEM_SHARED((NP, D), jnp.float32),
        ],
    )
    def deg(dstp_hbm, zn_hbm, ones_hbm, deg_out,
            dst_v, st_v, dacc):
        cid = lax.axis_index("c")
        sid = lax.axis_index("s")
        wid = cid * NS + sid

        rows0 = sid * RPT
        pltpu.sync_copy(zn_hbm, st_v)
        for off, sz in _stage_blocks(RPT):
            pltpu.sync_copy(st_v.at[pl.ds(0, sz)],
                            dacc.at[pl.ds(rows0 + off, sz)])
        pltpu.sync_copy(ones_hbm, st_v)
        plsc.subcore_barrier()

        def chunk(g, carry):
            base = (wid * CH + g) * K
            pltpu.sync_copy(dstp_hbm.at[pl.ds(base, K)], dst_v)
            pltpu.sync_copy(st_v, dacc.at[dst_v], add=True)
            return carry

        lax.fori_loop(0, CH, chunk, 0)
        plsc.subcore_barrier()

        out0 = cid * NP + rows0
        for off, sz in _stage_blocks(RPT):
            pltpu.sync_copy(dacc.at[pl.ds(rows0 + off, sz)],
                            st_v.at[pl.ds(0, sz)])
            pltpu.sync_copy(st_v.at[pl.ds(0, sz)],
                            deg_out.at[pl.ds(out0 + off, sz)])

    return deg


def _finalize_body(x_ref, nei_ref, deg_ref, w_ref, b_ref, o_ref, *, D):
    nei = nei_ref[0] + nei_ref[1]
    deg = deg_ref[0, :, 0:1] + deg_ref[1, :, 0:1]
    mean = nei / jnp.maximum(deg, 1.0)
    o_ref[...] = (
        jnp.dot(x_ref[...], w_ref[0:D, :], preferred_element_type=jnp.float32)
        + jnp.dot(mean, w_ref[D:, :], preferred_element_type=jnp.float32)
        + b_ref[...]
    )


def kernel(x, edge_index, weight, bias):
    N, D = x.shape
    E = edge_index.shape[1]
    OUT = weight.shape[1]

    CH = _cdiv(E, NW * K)
    EP = NW * K * CH
    NP = _cdiv(N + 1, 128) * 128

    src = edge_index[0]
    dst = edge_index[1]
    pad = EP - E
    srcp = jnp.concatenate([src, jnp.zeros((pad,), jnp.int32)])
    dstp = jnp.concatenate([dst, jnp.full((pad,), N, jnp.int32)])
    zn = jnp.zeros((K, D), jnp.float32)
    zn_g = jnp.zeros((KG, D), jnp.float32)
    ones_k = jnp.ones((K, D), jnp.float32)

    nei_flat = _sc_aggregate(N, D, EP, NP)(x, srcp, dstp, zn_g)
    deg_flat = _sc_degree(D, EP, NP)(dstp, zn, ones_k)
    nei_p = nei_flat.reshape(NC, NP, D)
    deg_p = deg_flat.reshape(NC, NP, D)

    BR = 2000
    out = pl.pallas_call(
        functools.partial(_finalize_body, D=D),
        grid=(N // BR,),
        in_specs=[
            pl.BlockSpec((BR, D), lambda i: (i, 0)),
            pl.BlockSpec((NC, BR, D), lambda i: (0, i, 0)),
            pl.BlockSpec((NC, BR, D), lambda i: (0, i, 0)),
            pl.BlockSpec((2 * D, OUT), lambda i: (0, 0)),
            pl.BlockSpec((1, OUT), lambda i: (0, 0)),
        ],
        out_specs=pl.BlockSpec((BR, OUT), lambda i: (i, 0)),
        out_shape=jax.ShapeDtypeStruct((N, OUT), jnp.float32),
    )(x, nei_p, deg_p, weight, bias.reshape(1, OUT))
    return out
